# baseline (speedup 1.0000x reference)
import functools

import jax
import jax.numpy as jnp
from jax import lax
from jax.experimental import pallas as pl
from jax.experimental.pallas import tpu as pltpu
from jax.experimental.pallas import tpu_sc as plsc

N = 10000
E = 320000
F = 128
H = 32
C = 64
PROP_RANGE = 3
ALPHA = 1.0
BETA = 0.5

NC = 2
NS = 16
NW = NC * NS

CHUNK = 128
CPT = 80
E_PAD = NW * CPT * CHUNK
NPAD = 10112
ZROWS = NPAD // NS


def _sc_scatter_body(p_hbm, rows_hbm, cols_hbm, zeros_hbm, out_hbm,
                     ridx, cidx, gbuf0, gbuf1, acc, sem0, sem1):
    c = lax.axis_index("c")
    s = lax.axis_index("s")
    w = c * NS + s

    with jax.named_scope("sc_init"):
        pltpu.sync_copy(zeros_hbm.at[pl.ds(s * ZROWS, ZROWS)],
                        acc.at[pl.ds(s * ZROWS, ZROWS)])

        base = w * CPT
        pltpu.sync_copy(rows_hbm.at[pl.ds(base, CPT)], ridx)
        pltpu.sync_copy(cols_hbm.at[pl.ds(base, CPT)], cidx)
        plsc.subcore_barrier()

    with jax.named_scope("sc_edges"):
        pltpu.async_copy(p_hbm.at[cidx.at[0]], gbuf0, sem0)
        pltpu.async_copy(p_hbm.at[cidx.at[1]], gbuf1, sem1)

        def pair(i, carry):
            j = 2 * i
            pltpu.make_async_copy(p_hbm.at[cidx.at[j]], gbuf0, sem0).wait()
            pltpu.sync_copy(gbuf0, acc.at[ridx.at[j]], add=True)
            pltpu.async_copy(p_hbm.at[cidx.at[j + 2]], gbuf0, sem0)
            pltpu.make_async_copy(p_hbm.at[cidx.at[j + 1]], gbuf1, sem1).wait()
            pltpu.sync_copy(gbuf1, acc.at[ridx.at[j + 1]], add=True)
            pltpu.async_copy(p_hbm.at[cidx.at[j + 3]], gbuf1, sem1)
            return carry

        lax.fori_loop(0, CPT // 2 - 1, pair, 0, unroll=1)

        j = CPT - 2
        pltpu.make_async_copy(p_hbm.at[cidx.at[j]], gbuf0, sem0).wait()
        pltpu.sync_copy(gbuf0, acc.at[ridx.at[j]], add=True)
        pltpu.make_async_copy(p_hbm.at[cidx.at[j + 1]], gbuf1, sem1).wait()
        pltpu.sync_copy(gbuf1, acc.at[ridx.at[j + 1]], add=True)

        plsc.subcore_barrier()

    with jax.named_scope("sc_out"):
        pltpu.sync_copy(acc.at[pl.ds(s * ZROWS, ZROWS)],
                        out_hbm.at[c, pl.ds(s * ZROWS, ZROWS)])


_sc_scatter = pl.kernel(
    _sc_scatter_body,
    out_type=jax.ShapeDtypeStruct((NC, NPAD, C), jnp.float32),
    mesh=plsc.VectorSubcoreMesh(core_axis_name="c", subcore_axis_name="s",
                                num_cores=NC, num_subcores=NS),
    scratch_types=[
        pltpu.VMEM((CPT, CHUNK), jnp.int32),
        pltpu.VMEM((CPT, CHUNK), jnp.int32),
        pltpu.VMEM((CHUNK, C), jnp.float32),
        pltpu.VMEM((CHUNK, C), jnp.float32),
        pltpu.VMEM_SHARED((NPAD, C), jnp.float32),
        pltpu.SemaphoreType.DMA,
        pltpu.SemaphoreType.DMA,
    ],
    compiler_params=pltpu.CompilerParams(use_tc_tiling_on_sc=False),
)


ROWS_BLK = 2000


def _mlp_body(x_ref, w1_ref, b1_ref, w2_ref, b2_ref, out_ref):
    h = jnp.dot(x_ref[...], w1_ref[...], preferred_element_type=jnp.float32)
    h = jnp.maximum(h + b1_ref[...], 0.0)
    lg = jnp.dot(h, w2_ref[...], preferred_element_type=jnp.float32)
    lg = lg + b2_ref[...]
    e = jnp.exp(lg - jnp.max(lg, axis=-1, keepdims=True))
    out_ref[...] = e / jnp.sum(e, axis=-1, keepdims=True)


def _mlp(x, W1, b1, W2, b2):
    return pl.pallas_call(
        _mlp_body,
        grid=(N // ROWS_BLK,),
        in_specs=[
            pl.BlockSpec((ROWS_BLK, F), lambda i: (i, 0)),
            pl.BlockSpec((F, H), lambda i: (0, 0)),
            pl.BlockSpec((1, H), lambda i: (0, 0)),
            pl.BlockSpec((H, C), lambda i: (0, 0)),
            pl.BlockSpec((1, C), lambda i: (0, 0)),
        ],
        out_specs=pl.BlockSpec((ROWS_BLK, C), lambda i: (i, 0)),
        out_shape=jax.ShapeDtypeStruct((N, C), jnp.float32),
    )(x, W1, b1.reshape(1, H), W2, b2.reshape(1, C))


def _hop_body(parts_ref, p_ref, y_ref, pnew_ref, ynew_ref):
    t = parts_ref[0] + parts_ref[1] + p_ref[...]
    t = jax.nn.sigmoid(ALPHA * t + BETA)
    pnew_ref[...] = t
    e = jnp.exp(t - jnp.max(t, axis=-1, keepdims=True))
    ynew_ref[...] = y_ref[...] + e / jnp.sum(e, axis=-1, keepdims=True)


def _hop(parts, p, y):
    return pl.pallas_call(
        _hop_body,
        grid=(N // ROWS_BLK,),
        in_specs=[
            pl.BlockSpec((NC, ROWS_BLK, C), lambda i: (0, i, 0)),
            pl.BlockSpec((ROWS_BLK, C), lambda i: (i, 0)),
            pl.BlockSpec((ROWS_BLK, C), lambda i: (i, 0)),
        ],
        out_specs=[
            pl.BlockSpec((ROWS_BLK, C), lambda i: (i, 0)),
            pl.BlockSpec((ROWS_BLK, C), lambda i: (i, 0)),
        ],
        out_shape=[
            jax.ShapeDtypeStruct((N, C), jnp.float32),
            jax.ShapeDtypeStruct((N, C), jnp.float32),
        ],
    )(parts, p, y)


def kernel(x, edge_index, W1, b1, W2, b2):
    rows = edge_index[0]
    cols = edge_index[1]
    pad = E_PAD - E
    trash = N + (jnp.arange(pad, dtype=jnp.int32) % (NPAD - N))
    rows_p = jnp.concatenate([rows, trash])
    cols_p = jnp.concatenate([cols, jnp.zeros((pad,), jnp.int32)])
    rows2d = rows_p.reshape(NW * CPT, CHUNK)
    cols2d = cols_p.reshape(NW * CPT, CHUNK)
    zeros_pad = jnp.zeros((NPAD, C), jnp.float32)

    p = _mlp(x, W1, b1, W2, b2)
    y = jnp.zeros((N, C), jnp.float32)
    for _ in range(PROP_RANGE):
        parts = _sc_scatter(p, rows2d, cols2d, zeros_pad)
        p, y = _hop(parts, p, y)
    return y


# speedup vs baseline: 2.1249x; 2.1249x over previous
import functools

import jax
import jax.numpy as jnp
from jax import lax
from jax.experimental import pallas as pl
from jax.experimental.pallas import tpu as pltpu
from jax.experimental.pallas import tpu_sc as plsc

N = 10000
E = 320000
F = 128
H = 32
C = 64
PROP_RANGE = 3
ALPHA = 1.0
BETA = 0.5

NC = 2
NS = 16
NW = NC * NS

CHUNK = 128
CPT = 80
E_PAD = NW * CPT * CHUNK
NPAD = 10112
ZROWS = NPAD // NS


def _sc_scatter_body(p_hbm, rows_hbm, cols_hbm, zeros_hbm, out_hbm,
                     ridx, cidx, gbuf0, gbuf1, acc, ptab, sem0, sem1):
    c = lax.axis_index("c")
    s = lax.axis_index("s")
    w = c * NS + s

    with jax.named_scope("sc_init"):
        pltpu.sync_copy(zeros_hbm.at[pl.ds(s * ZROWS, ZROWS)],
                        acc.at[pl.ds(s * ZROWS, ZROWS)])
        pltpu.sync_copy(p_hbm.at[pl.ds(s * ZROWS, ZROWS)],
                        ptab.at[pl.ds(s * ZROWS, ZROWS)])

        base = w * CPT
        pltpu.sync_copy(rows_hbm.at[pl.ds(base, CPT)], ridx)
        pltpu.sync_copy(cols_hbm.at[pl.ds(base, CPT)], cidx)
        plsc.subcore_barrier()

    with jax.named_scope("sc_edges"):
        pltpu.async_copy(ptab.at[cidx.at[0]], gbuf0, sem0)
        pltpu.async_copy(ptab.at[cidx.at[1]], gbuf1, sem1)

        def pair(i, carry):
            j = 2 * i
            pltpu.make_async_copy(ptab.at[cidx.at[j]], gbuf0, sem0).wait()
            pltpu.sync_copy(gbuf0, acc.at[ridx.at[j]], add=True)
            pltpu.async_copy(ptab.at[cidx.at[j + 2]], gbuf0, sem0)
            pltpu.make_async_copy(ptab.at[cidx.at[j + 1]], gbuf1, sem1).wait()
            pltpu.sync_copy(gbuf1, acc.at[ridx.at[j + 1]], add=True)
            pltpu.async_copy(ptab.at[cidx.at[j + 3]], gbuf1, sem1)
            return carry

        lax.fori_loop(0, CPT // 2 - 1, pair, 0, unroll=1)

        j = CPT - 2
        pltpu.make_async_copy(ptab.at[cidx.at[j]], gbuf0, sem0).wait()
        pltpu.sync_copy(gbuf0, acc.at[ridx.at[j]], add=True)
        pltpu.make_async_copy(ptab.at[cidx.at[j + 1]], gbuf1, sem1).wait()
        pltpu.sync_copy(gbuf1, acc.at[ridx.at[j + 1]], add=True)

        plsc.subcore_barrier()

    with jax.named_scope("sc_out"):
        pltpu.sync_copy(acc.at[pl.ds(s * ZROWS, ZROWS)],
                        out_hbm.at[c, pl.ds(s * ZROWS, ZROWS)])


_sc_scatter = pl.kernel(
    _sc_scatter_body,
    out_type=jax.ShapeDtypeStruct((NC, NPAD, C), jnp.float32),
    mesh=plsc.VectorSubcoreMesh(core_axis_name="c", subcore_axis_name="s",
                                num_cores=NC, num_subcores=NS),
    scratch_types=[
        pltpu.VMEM((CPT, CHUNK), jnp.int32),
        pltpu.VMEM((CPT, CHUNK), jnp.int32),
        pltpu.VMEM((CHUNK, C), jnp.float32),
        pltpu.VMEM((CHUNK, C), jnp.float32),
        pltpu.VMEM_SHARED((NPAD, C), jnp.float32),
        pltpu.VMEM_SHARED((NPAD, C), jnp.float32),
        pltpu.SemaphoreType.DMA,
        pltpu.SemaphoreType.DMA,
    ],
    compiler_params=pltpu.CompilerParams(use_tc_tiling_on_sc=False),
)


ROWS_BLK = 2000


def _mlp_body(x_ref, w1_ref, b1_ref, w2_ref, b2_ref, out_ref):
    h = jnp.dot(x_ref[...], w1_ref[...], preferred_element_type=jnp.float32)
    h = jnp.maximum(h + b1_ref[...], 0.0)
    lg = jnp.dot(h, w2_ref[...], preferred_element_type=jnp.float32)
    lg = lg + b2_ref[...]
    e = jnp.exp(lg - jnp.max(lg, axis=-1, keepdims=True))
    out_ref[...] = e / jnp.sum(e, axis=-1, keepdims=True)


def _mlp(x, W1, b1, W2, b2):
    return pl.pallas_call(
        _mlp_body,
        grid=(N // ROWS_BLK,),
        in_specs=[
            pl.BlockSpec((ROWS_BLK, F), lambda i: (i, 0)),
            pl.BlockSpec((F, H), lambda i: (0, 0)),
            pl.BlockSpec((1, H), lambda i: (0, 0)),
            pl.BlockSpec((H, C), lambda i: (0, 0)),
            pl.BlockSpec((1, C), lambda i: (0, 0)),
        ],
        out_specs=pl.BlockSpec((ROWS_BLK, C), lambda i: (i, 0)),
        out_shape=jax.ShapeDtypeStruct((N, C), jnp.float32),
    )(x, W1, b1.reshape(1, H), W2, b2.reshape(1, C))


def _hop_body(parts_ref, p_ref, y_ref, pnew_ref, ynew_ref):
    t = parts_ref[0] + parts_ref[1] + p_ref[...]
    t = jax.nn.sigmoid(ALPHA * t + BETA)
    pnew_ref[...] = t
    e = jnp.exp(t - jnp.max(t, axis=-1, keepdims=True))
    ynew_ref[...] = y_ref[...] + e / jnp.sum(e, axis=-1, keepdims=True)


def _hop(parts, p, y):
    return pl.pallas_call(
        _hop_body,
        grid=(N // ROWS_BLK,),
        in_specs=[
            pl.BlockSpec((NC, ROWS_BLK, C), lambda i: (0, i, 0)),
            pl.BlockSpec((ROWS_BLK, C), lambda i: (i, 0)),
            pl.BlockSpec((ROWS_BLK, C), lambda i: (i, 0)),
        ],
        out_specs=[
            pl.BlockSpec((ROWS_BLK, C), lambda i: (i, 0)),
            pl.BlockSpec((ROWS_BLK, C), lambda i: (i, 0)),
        ],
        out_shape=[
            jax.ShapeDtypeStruct((N, C), jnp.float32),
            jax.ShapeDtypeStruct((N, C), jnp.float32),
        ],
    )(parts, p, y)


def kernel(x, edge_index, W1, b1, W2, b2):
    rows = edge_index[0]
    cols = edge_index[1]
    pad = E_PAD - E
    trash = N + (jnp.arange(pad, dtype=jnp.int32) % (NPAD - N))
    rows_p = jnp.concatenate([rows, trash])
    cols_p = jnp.concatenate([cols, jnp.zeros((pad,), jnp.int32)])
    rows2d = rows_p.reshape(NW * CPT, CHUNK)
    cols2d = cols_p.reshape(NW * CPT, CHUNK)
    zeros_pad = jnp.zeros((NPAD, C), jnp.float32)

    p = _mlp(x, W1, b1, W2, b2)
    y = jnp.zeros((N, C), jnp.float32)
    for _ in range(PROP_RANGE):
        p_pad = jnp.pad(p, ((0, NPAD - N), (0, 0)))
        parts = _sc_scatter(p_pad, rows2d, cols2d, zeros_pad)
        p, y = _hop(parts, p, y)
    return y
